# baseline (speedup 1.0000x reference)
import functools

import jax
import jax.numpy as jnp
from jax import lax
from jax.experimental import pallas as pl
from jax.experimental.pallas import tpu as pltpu
from jax.experimental.pallas import tpu_sc as plsc

N_NODES = 50000
INTRA_DEG = 64
INTER_DEG = 32
BATCH = 16384
N_SAMPLES = 25
N_SHEETS = 10

NUM_CORES = 2
NUM_SUBCORES = 16
NUM_WORKERS = NUM_CORES * NUM_SUBCORES
B_PER_W = BATCH // NUM_WORKERS
CHUNK = 128
N_CHUNKS = B_PER_W // CHUNK
OUT_W = N_SAMPLES + N_SHEETS

COLS_INTRA = (3, 59, 0, 41, 20, 31, 6, 8, 45, 29, 61, 39, 24, 5, 62,
              14, 1, 53, 36, 51, 60, 33, 56, 26, 15)
COLS_INTER = (18, 8, 2, 6, 0, 19, 25, 11, 27, 30)

_MESH = plsc.VectorSubcoreMesh(core_axis_name="c", subcore_axis_name="s")


def _full16(v):
    return jnp.full((16,), v, jnp.int32)


@functools.partial(
    pl.kernel,
    out_type=jax.ShapeDtypeStruct((BATCH, OUT_W), jnp.int32),
    mesh=_MESH,
    scratch_types=[
        pltpu.VMEM((B_PER_W,), jnp.int32),
        pltpu.VMEM((B_PER_W, INTRA_DEG), jnp.int32),
        pltpu.VMEM((B_PER_W, INTER_DEG), jnp.int32),
        pltpu.VMEM((B_PER_W, OUT_W), jnp.int32),
        pltpu.SemaphoreType.DMA,
    ],
    compiler_params=pltpu.CompilerParams(
        needs_layout_passes=False, use_tc_tiling_on_sc=False),
)
def _sc_sampler(intra_hbm, inter_hbm, ids_hbm, out_hbm,
                idx_v, rows_i, rows_t, out_v, sem):
    wid = lax.axis_index("s") * NUM_CORES + lax.axis_index("c")
    base = wid * B_PER_W

    pltpu.sync_copy(ids_hbm.at[pl.ds(base, B_PER_W)], idx_v)

    copies = []
    for k in range(N_CHUNKS):
        sl = pl.ds(k * CHUNK, CHUNK)
        copies.append(pltpu.async_copy(intra_hbm.at[idx_v.at[sl]], rows_i.at[sl], sem))
        copies.append(pltpu.async_copy(inter_hbm.at[idx_v.at[sl]], rows_t.at[sl], sem))
    for c in copies:
        c.wait()

    iota = lax.iota(jnp.int32, 16)

    def body(g, carry):
        rvec = g * jnp.int32(16) + iota
        for j, c in enumerate(COLS_INTRA):
            v = plsc.load_gather(rows_i, [rvec, _full16(c)])
            plsc.store_scatter(out_v, [rvec, _full16(j)], v)
        for j, c in enumerate(COLS_INTER):
            v = plsc.load_gather(rows_t, [rvec, _full16(c)])
            plsc.store_scatter(out_v, [rvec, _full16(N_SAMPLES + j)], v)
        return carry

    lax.fori_loop(jnp.int32(0), jnp.int32(B_PER_W // 16), body, jnp.int32(0))

    pltpu.sync_copy(out_v, out_hbm.at[pl.ds(base, B_PER_W)])


def kernel(intra_adj_info, inter_adj_info, ids, num_samples, num_sheets):
    del num_samples, num_sheets
    intra_flat = intra_adj_info.reshape(-1).astype(jnp.int32)
    inter_flat = inter_adj_info.reshape(-1).astype(jnp.int32)
    intra_flat, inter_flat = lax.optimization_barrier((intra_flat, inter_flat))
    intra32 = intra_flat.reshape(N_NODES, INTRA_DEG)
    inter32 = inter_flat.reshape(N_NODES, INTER_DEG)
    ids32 = ids.astype(jnp.int32)
    out32 = _sc_sampler(intra32, inter32, ids32)
    return out32.astype(intra_adj_info.dtype)


# speedup vs baseline: 1.2465x; 1.2465x over previous
import functools

import jax
import jax.numpy as jnp
from jax import lax
from jax.experimental import pallas as pl
from jax.experimental.pallas import tpu as pltpu
from jax.experimental.pallas import tpu_sc as plsc

N_NODES = 50000
INTRA_DEG = 64
INTER_DEG = 32
BATCH = 16384
N_SAMPLES = 25
N_SHEETS = 10

NUM_CORES = 2
NUM_SUBCORES = 16
NUM_WORKERS = NUM_CORES * NUM_SUBCORES
B_PER_W = BATCH // NUM_WORKERS
CHUNK = 128
N_CHUNKS = B_PER_W // CHUNK
OUT_W = N_SAMPLES + N_SHEETS

COLS_INTRA = (3, 59, 0, 41, 20, 31, 6, 8, 45, 29, 61, 39, 24, 5, 62,
              14, 1, 53, 36, 51, 60, 33, 56, 26, 15)
COLS_INTER = (18, 8, 2, 6, 0, 19, 25, 11, 27, 30)

_MESH = plsc.VectorSubcoreMesh(core_axis_name="c", subcore_axis_name="s")


def _full16(v):
    return jnp.full((16,), v, jnp.int32)


def _make_gather(deg, cols):
    out_w = len(cols)

    @functools.partial(
        pl.kernel,
        out_type=jax.ShapeDtypeStruct((BATCH, out_w), jnp.int32),
        mesh=_MESH,
        scratch_types=[
            pltpu.VMEM((B_PER_W,), jnp.int32),
            pltpu.VMEM((B_PER_W, deg), jnp.int32),
            pltpu.VMEM((B_PER_W, out_w), jnp.int32),
            pltpu.SemaphoreType.DMA,
        ],
        compiler_params=pltpu.CompilerParams(
            needs_layout_passes=False, use_tc_tiling_on_sc=False),
    )
    def _gather(tab_hbm, ids_hbm, out_hbm, idx_v, rows_v, out_v, sem):
        wid = lax.axis_index("s") * NUM_CORES + lax.axis_index("c")
        base = wid * B_PER_W

        pltpu.sync_copy(ids_hbm.at[pl.ds(base, B_PER_W)], idx_v)

        copies = []
        for k in range(N_CHUNKS):
            sl = pl.ds(k * CHUNK, CHUNK)
            copies.append(pltpu.async_copy(tab_hbm.at[idx_v.at[sl]],
                                           rows_v.at[sl], sem))
        for c in copies:
            c.wait()

        iota = lax.iota(jnp.int32, 16)

        def body(g, carry):
            rvec = g * jnp.int32(16) + iota
            for j, c in enumerate(cols):
                v = plsc.load_gather(rows_v, [rvec, _full16(c)])
                plsc.store_scatter(out_v, [rvec, _full16(j)], v)
            return carry

        lax.fori_loop(jnp.int32(0), jnp.int32(B_PER_W // 16), body,
                      jnp.int32(0))

        pltpu.sync_copy(out_v, out_hbm.at[pl.ds(base, B_PER_W)])

    return _gather


_gather_intra = _make_gather(INTRA_DEG, COLS_INTRA)
_gather_inter = _make_gather(INTER_DEG, COLS_INTER)


def kernel(intra_adj_info, inter_adj_info, ids, num_samples, num_sheets):
    del num_samples, num_sheets
    ids32 = ids.astype(jnp.int32)

    intra_flat = intra_adj_info.reshape(-1).astype(jnp.int32)
    intra_flat = lax.optimization_barrier(intra_flat)
    out_a = _gather_intra(intra_flat.reshape(N_NODES, INTRA_DEG), ids32)

    inter_flat = inter_adj_info.reshape(-1).astype(jnp.int32)
    inter_flat = lax.optimization_barrier(inter_flat)
    out_b = _gather_inter(inter_flat.reshape(N_NODES, INTER_DEG), ids32)

    dt = intra_adj_info.dtype
    return jnp.concatenate([out_a.astype(dt), out_b.astype(dt)], axis=1)
